# baseline (speedup 1.0000x reference)
import functools

import jax
import jax.numpy as jnp
from jax.experimental import pallas as pl
from jax.experimental.pallas import tpu as pltpu

D_MODEL = 4096
NUM_EXPERTS = 64
TOKENS = 4 * 2048
BLOCK_T = 512
BLOCK_K = 1024
GRID_T = TOKENS // BLOCK_T
GRID_K = D_MODEL // BLOCK_K


def _router_body(x_ref, wt_ref, b_ref, disp_ref, counts_ref, loss_ref, acc_ref):
    t = pl.program_id(0)
    k = pl.program_id(1)
    part = jnp.dot(x_ref[...], wt_ref[...], preferred_element_type=jnp.float32)

    @pl.when(k == 0)
    def _():
        acc_ref[...] = part

    @pl.when(k > 0)
    def _():
        acc_ref[...] = acc_ref[...] + part

    @pl.when(k == GRID_K - 1)
    def _():
        logits = acc_ref[...] + b_ref[...]
        idx = jnp.argmax(logits, axis=1)
        lanes = jax.lax.broadcasted_iota(jnp.int32, (BLOCK_T, NUM_EXPERTS), 1)
        onehot = (lanes == idx[:, None]).astype(jnp.float32)
        disp_ref[...] = onehot
        partial = jnp.sum(onehot, axis=0, keepdims=True)

        @pl.when(t == 0)
        def _():
            counts_ref[...] = partial

        @pl.when(t > 0)
        def _():
            counts_ref[...] = counts_ref[...] + partial

        @pl.when(t == GRID_T - 1)
        def _():
            counts = counts_ref[...]
            total = jnp.maximum(jnp.sum(counts), 1.0)
            lb = counts * (NUM_EXPERTS / total)
            loss_ref[...] = jnp.mean((lb - 1.0) ** 2).reshape(1, 1)


@functools.partial(jax.jit, static_argnames=())
def kernel(x, W, b):
    xf = x.reshape(TOKENS, D_MODEL)
    wt = W.T
    b2 = b.reshape(1, NUM_EXPERTS)
    disp, counts, loss = pl.pallas_call(
        _router_body,
        grid=(GRID_T, GRID_K),
        in_specs=[
            pl.BlockSpec((BLOCK_T, BLOCK_K), lambda t, k: (t, k)),
            pl.BlockSpec((BLOCK_K, NUM_EXPERTS), lambda t, k: (k, 0)),
            pl.BlockSpec((1, NUM_EXPERTS), lambda t, k: (0, 0)),
        ],
        out_specs=[
            pl.BlockSpec((BLOCK_T, NUM_EXPERTS), lambda t, k: (t, 0)),
            pl.BlockSpec((1, NUM_EXPERTS), lambda t, k: (0, 0)),
            pl.BlockSpec((1, 1), lambda t, k: (0, 0)),
        ],
        out_shape=[
            jax.ShapeDtypeStruct((TOKENS, NUM_EXPERTS), jnp.float32),
            jax.ShapeDtypeStruct((1, NUM_EXPERTS), jnp.float32),
            jax.ShapeDtypeStruct((1, 1), jnp.float32),
        ],
        scratch_shapes=[
            pltpu.VMEM((BLOCK_T, NUM_EXPERTS), jnp.float32),
        ],
    )(xf, wt, b2)
    dispatch = disp.reshape(x.shape[0], x.shape[1], NUM_EXPERTS)
    expert_counts = counts.reshape(NUM_EXPERTS)
    load_balance_loss = loss[0, 0]
    return dispatch, dispatch, expert_counts, load_balance_loss, expert_counts


# speedup vs baseline: 1.6879x; 1.6879x over previous
import functools

import jax
import jax.numpy as jnp
from jax.experimental import pallas as pl

D_MODEL = 4096
NUM_EXPERTS = 64
TOKENS = 4 * 2048
BLOCK_T = 512
SUB_T = BLOCK_T // 2
GRID = TOKENS // BLOCK_T


def _router_body(xa_ref, xb_ref, wt_ref, b_ref, disp_ref, counts_ref, loss_ref):
    step = pl.program_id(0)
    partials = []
    for sub, ref in ((0, xa_ref), (1, xb_ref)):
        logits = jnp.dot(ref[...], wt_ref[...], preferred_element_type=jnp.float32)
        logits = logits + b_ref[...]
        idx = jnp.argmax(logits, axis=1)
        lanes = jax.lax.broadcasted_iota(jnp.int32, (SUB_T, NUM_EXPERTS), 1)
        onehot = (lanes == idx[:, None]).astype(jnp.float32)
        disp_ref[pl.ds(sub * SUB_T, SUB_T), :] = onehot
        partials.append(jnp.sum(onehot, axis=0, keepdims=True))
    partial = partials[0] + partials[1]

    @pl.when(step == 0)
    def _():
        counts_ref[...] = partial

    @pl.when(step > 0)
    def _():
        counts_ref[...] = counts_ref[...] + partial

    @pl.when(step == GRID - 1)
    def _():
        counts = counts_ref[...]
        total = jnp.maximum(jnp.sum(counts), 1.0)
        lb = counts * (NUM_EXPERTS / total)
        loss_ref[...] = jnp.mean((lb - 1.0) ** 2).reshape(1, 1)


@functools.partial(jax.jit, static_argnames=())
def kernel(x, W, b):
    xf = x.reshape(TOKENS, D_MODEL)
    wt = W.T
    b2 = b.reshape(1, NUM_EXPERTS)
    disp, counts, loss = pl.pallas_call(
        _router_body,
        grid=(GRID,),
        in_specs=[
            pl.BlockSpec((SUB_T, D_MODEL), lambda i: (2 * i, 0)),
            pl.BlockSpec((SUB_T, D_MODEL), lambda i: (2 * i + 1, 0)),
            pl.BlockSpec((D_MODEL, NUM_EXPERTS), lambda i: (0, 0)),
            pl.BlockSpec((1, NUM_EXPERTS), lambda i: (0, 0)),
        ],
        out_specs=[
            pl.BlockSpec((BLOCK_T, NUM_EXPERTS), lambda i: (i, 0)),
            pl.BlockSpec((1, NUM_EXPERTS), lambda i: (0, 0)),
            pl.BlockSpec((1, 1), lambda i: (0, 0)),
        ],
        out_shape=[
            jax.ShapeDtypeStruct((TOKENS, NUM_EXPERTS), jnp.float32),
            jax.ShapeDtypeStruct((1, NUM_EXPERTS), jnp.float32),
            jax.ShapeDtypeStruct((1, 1), jnp.float32),
        ],
    )(xf, xf, wt, b2)
    dispatch = disp.reshape(x.shape[0], x.shape[1], NUM_EXPERTS)
    expert_counts = counts.reshape(NUM_EXPERTS)
    load_balance_loss = loss[0, 0]
    return dispatch, dispatch, expert_counts, load_balance_loss, expert_counts
